# baseline (speedup 1.0000x reference)
import functools

import jax
import jax.numpy as jnp
from jax import lax
from jax.experimental import pallas as pl
from jax.experimental.pallas import tpu as pltpu
from jax.experimental.pallas import tpu_sc as plsc

BATCH = 16384
HIST = 50
DIM = 32
NUM_CORES = 2
NUM_SUBCORES = 16
NW = NUM_CORES * NUM_SUBCORES
BPW = BATCH // NW
NLIST = BPW // 128
NK = BPW // 16

_mesh = plsc.VectorSubcoreMesh(core_axis_name="c", subcore_axis_name="s")

NEMB = 1000000
TCH = 512
NCH = (NEMB + TCH - 1) // TCH
TSLOT = 2 * ((NCH + 2 * NW - 1) // (2 * NW))


@functools.partial(
    pl.kernel,
    mesh=_mesh,
    out_type=jax.ShapeDtypeStruct((NEMB, DIM), jnp.float32),
    scratch_types=[
        pltpu.VMEM((2, DIM, TCH), jnp.float32),
        pltpu.VMEM((2, TCH, DIM), jnp.float32),
        pltpu.SemaphoreType.DMA,
        pltpu.SemaphoreType.DMA,
        pltpu.SemaphoreType.DMA,
    ],
    compiler_params=pltpu.CompilerParams(
        use_tc_tiling_on_sc=False, needs_layout_passes=False
    ),
)
def _transpose_kernel(wt_hbm, out_hbm, in_v, out_v, si0, si1, so):
    wid = lax.axis_index("s") * NUM_CORES + lax.axis_index("c")
    lanes = lax.iota(jnp.int32, 16)
    colvecs = [
        ((j + lanes) & 15) + h * 16 for h in range(DIM // 16) for j in range(16)
    ]

    def v0_of(s):
        cid = s * NW + wid
        v0 = jnp.where(cid * TCH + TCH > NEMB, NEMB - TCH, cid * TCH)
        return cid, pl.multiple_of(v0, 8)

    def in_copy(s, buf, sem):
        _, v0 = v0_of(s)
        return pltpu.make_async_copy(
            wt_hbm.at[:, pl.ds(v0, TCH)], in_v.at[buf], sem
        )

    def transpose_block(buf):
        iref = in_v.at[buf]
        oref = out_v.at[buf]

        def btbody(bt, carry):
            rvs = [bt * 64 + u * 16 + lanes for u in range(4)]
            for cv in colvecs:
                vs = [plsc.load_gather(iref, [cv, rv]) for rv in rvs]
                for rv, v in zip(rvs, vs):
                    plsc.store_scatter(oref, [rv, cv], v)
            return carry

        lax.fori_loop(0, TCH // 64, btbody, 0)

    in_copy(0, 0, si0).start()

    def body(g, carry):
        for buf, sem_in in ((0, si0), (1, si1)):
            s = 2 * g + buf
            cid, v0 = v0_of(s)

            @pl.when(cid < NCH)
            def _():
                in_copy(s, buf, sem_in).wait()

                @pl.when(s + 1 < TSLOT)
                def _():
                    nid, _ = v0_of(s + 1)

                    @pl.when(nid < NCH)
                    def _():
                        in_copy(s + 1, 1 - buf, si1 if buf == 0 else si0).start()

                transpose_block(buf)
                pltpu.async_copy(
                    out_v.at[buf], out_hbm.at[pl.ds(v0, TCH)], so
                ).wait()

        return carry

    lax.fori_loop(0, TSLOT // 2, body, 0)


@functools.partial(
    pl.kernel,
    mesh=_mesh,
    out_type=jax.ShapeDtypeStruct((HIST, DIM, BATCH), jnp.float32),
    scratch_types=[
        pltpu.VMEM((HIST, BPW), jnp.int32),
        pltpu.VMEM((2, BPW, DIM), jnp.float32),
        pltpu.VMEM((2, DIM, BPW), jnp.float32),
        pltpu.SemaphoreType.DMA,
        pltpu.SemaphoreType.DMA,
        pltpu.SemaphoreType.DMA,
        pltpu.SemaphoreType.DMA,
    ],
    compiler_params=pltpu.CompilerParams(
        use_tc_tiling_on_sc=False, needs_layout_passes=False
    ),
)
def _gather_kernel(idx_hbm, table_hbm, out_hbm, idx_v, rows_v, tbuf_v,
                   sg0, sg1, so0, so1):
    wid = lax.axis_index("s") * NUM_CORES + lax.axis_index("c")
    wb0 = pl.multiple_of(wid * BPW, BPW)
    lanes = lax.iota(jnp.int32, 16)

    pltpu.sync_copy(idx_hbm.at[:, pl.ds(wb0, BPW)], idx_v)

    def gather_copies(l, buf, sem):
        return [
            pltpu.make_async_copy(
                table_hbm.at[idx_v.at[l].at[pl.ds(j * 128, 128)]],
                rows_v.at[buf].at[pl.ds(j * 128, 128)],
                sem,
            )
            for j in range(NLIST)
        ]

    def fire_gathers(l, buf, sem):
        for c in gather_copies(l, buf, sem):
            c.start()

    def wait_gathers(l, buf, sem):
        for c in gather_copies(l, buf, sem):
            c.wait()

    colvecs = [
        ((j + lanes) & 15) + h * 16 for h in range(DIM // 16) for j in range(16)
    ]

    def transpose(buf):
        rows_ref = rows_v.at[buf]
        tref = tbuf_v.at[buf]

        def btbody(bt, carry):
            rvs = [bt * 64 + u * 16 + lanes for u in range(4)]
            for cv in colvecs:
                vs = [plsc.load_gather(rows_ref, [rv, cv]) for rv in rvs]
                for rv, v in zip(rvs, vs):
                    plsc.store_scatter(tref, [cv, rv], v)
            return carry

        lax.fori_loop(0, NK // 4, btbody, 0)

    def out_copy(l, buf, sem):
        return pltpu.make_async_copy(
            tbuf_v.at[buf],
            out_hbm.at[l].at[:, pl.ds(wb0, BPW)],
            sem,
        )

    fire_gathers(0, 0, sg0)

    def body(i, carry):
        l0 = 2 * i
        l1 = 2 * i + 1

        wait_gathers(l0, 0, sg0)
        fire_gathers(l1, 1, sg1)

        @pl.when(i > 0)
        def _():
            out_copy(l0 - 2, 0, so0).wait()

        transpose(0)
        out_copy(l0, 0, so0).start()

        wait_gathers(l1, 1, sg1)

        @pl.when(i < HIST // 2 - 1)
        def _():
            fire_gathers(l0 + 2, 0, sg0)

        @pl.when(i > 0)
        def _():
            out_copy(l1 - 2, 1, so1).wait()

        transpose(1)
        out_copy(l1, 1, so1).start()
        return carry

    lax.fori_loop(0, HIST // 2, body, 0)
    out_copy(HIST - 2, 0, so0).wait()
    out_copy(HIST - 1, 1, so1).wait()


def kernel(input, weight):
    idx_t = input.T.astype(jnp.int32)
    w_rows = _transpose_kernel(weight.T)
    out = _gather_kernel(idx_t, w_rows)
    return jnp.transpose(out, (2, 0, 1))


# speedup vs baseline: 3.9929x; 3.9929x over previous
import functools

import jax
import jax.numpy as jnp
from jax import lax
from jax.experimental import pallas as pl
from jax.experimental.pallas import tpu as pltpu
from jax.experimental.pallas import tpu_sc as plsc

BATCH = 16384
HIST = 50
DIM = 32
NUM_CORES = 2
NUM_SUBCORES = 16
NW = NUM_CORES * NUM_SUBCORES
BPW = BATCH // NW
NLIST = BPW // 128
NK = BPW // 16

_mesh = plsc.VectorSubcoreMesh(core_axis_name="c", subcore_axis_name="s")


@functools.partial(
    pl.kernel,
    mesh=_mesh,
    out_type=jax.ShapeDtypeStruct((HIST, DIM, BATCH), jnp.float32),
    scratch_types=[
        pltpu.VMEM((HIST, BPW), jnp.int32),
        pltpu.VMEM((2, BPW, DIM), jnp.float32),
        pltpu.VMEM((2, DIM, BPW), jnp.float32),
        pltpu.SemaphoreType.DMA,
        pltpu.SemaphoreType.DMA,
        pltpu.SemaphoreType.DMA,
        pltpu.SemaphoreType.DMA,
    ],
    compiler_params=pltpu.CompilerParams(
        use_tc_tiling_on_sc=False, needs_layout_passes=False
    ),
)
def _gather_kernel(idx_hbm, table_hbm, out_hbm, idx_v, rows_v, tbuf_v,
                   sg0, sg1, so0, so1):
    wid = lax.axis_index("s") * NUM_CORES + lax.axis_index("c")
    wb0 = pl.multiple_of(wid * BPW, BPW)
    lanes = lax.iota(jnp.int32, 16)

    pltpu.sync_copy(idx_hbm.at[:, pl.ds(wb0, BPW)], idx_v)

    def gather_copies(l, buf, sem):
        return [
            pltpu.make_async_copy(
                table_hbm.at[idx_v.at[l].at[pl.ds(j * 128, 128)]],
                rows_v.at[buf].at[pl.ds(j * 128, 128)],
                sem,
            )
            for j in range(NLIST)
        ]

    def fire_gathers(l, buf, sem):
        for c in gather_copies(l, buf, sem):
            c.start()

    def wait_gathers(l, buf, sem):
        for c in gather_copies(l, buf, sem):
            c.wait()

    colvecs = [
        ((j + lanes) & 15) + h * 16 for h in range(DIM // 16) for j in range(16)
    ]

    def transpose(buf):
        rows_ref = rows_v.at[buf]
        tref = tbuf_v.at[buf]

        def btbody(bt, carry):
            rvs = [bt * 64 + u * 16 + lanes for u in range(4)]
            for cv in colvecs:
                vs = [plsc.load_gather(rows_ref, [rv, cv]) for rv in rvs]
                for rv, v in zip(rvs, vs):
                    plsc.store_scatter(tref, [cv, rv], v)
            return carry

        lax.fori_loop(0, NK // 4, btbody, 0)

    def out_copy(l, buf, sem):
        return pltpu.make_async_copy(
            tbuf_v.at[buf],
            out_hbm.at[l].at[:, pl.ds(wb0, BPW)],
            sem,
        )

    fire_gathers(0, 0, sg0)

    def body(i, carry):
        l0 = 2 * i
        l1 = 2 * i + 1

        wait_gathers(l0, 0, sg0)
        fire_gathers(l1, 1, sg1)

        @pl.when(i > 0)
        def _():
            out_copy(l0 - 2, 0, so0).wait()

        transpose(0)
        out_copy(l0, 0, so0).start()

        wait_gathers(l1, 1, sg1)

        @pl.when(i < HIST // 2 - 1)
        def _():
            fire_gathers(l0 + 2, 0, sg0)

        @pl.when(i > 0)
        def _():
            out_copy(l1 - 2, 1, so1).wait()

        transpose(1)
        out_copy(l1, 1, so1).start()
        return carry

    lax.fori_loop(0, HIST // 2, body, 0)
    out_copy(HIST - 2, 0, so0).wait()
    out_copy(HIST - 1, 1, so1).wait()


def kernel(input, weight):
    idx_t = input.T.astype(jnp.int32)
    out = _gather_kernel(idx_t, weight)
    return jnp.transpose(out, (2, 0, 1))
